# baseline (speedup 1.0000x reference)
import jax
import jax.numpy as jnp
from jax import lax
from jax.experimental import pallas as pl
from jax.experimental.pallas import tpu as pltpu
from jax.experimental.pallas import tpu_sc as plsc

_NC = 2
_NS = 16
_NW = _NC * _NS
_LANES = 16

_CHUNK = 128
_PITCH = 65


def _worker_id():
  return lax.axis_index("s") * _NC + lax.axis_index("c")


def _iota16():
  return lax.iota(jnp.int32, _LANES)


def _transpose_block(src, dst, n_rows):
  n_g = n_rows // _LANES
  rows = [_iota16() + g * _LANES for g in range(n_g)]

  @plsc.parallel_loop(0, 64, unroll=8)
  def _(d):
    dv = jnp.full((_LANES,), d, jnp.int32)
    for g in range(n_g):
      v = src[d, pl.ds(g * _LANES, _LANES)]
      plsc.store_scatter(dst, [rows[g], dv], v)


def _build_transpose(vocab, emb_dim):
  n_full = vocab // _CHUNK
  tail = vocab - n_full * _CHUNK
  iters = n_full // _NW + 1
  mesh = plsc.VectorSubcoreMesh(core_axis_name="c", subcore_axis_name="s")

  def body(tab_t, out_rm, buf0, buf1, buft, tbuf, tbuft, sem0, sem1):
    w = _worker_id()
    bufs = (buf0, buf1)
    sems = (sem0, sem1)

    def start(c, b):
      @pl.when(c < n_full)
      def _():
        pltpu.async_copy(tab_t.at[:, pl.ds(c * _CHUNK, _CHUNK)], bufs[b], sems[b])

    def finish(c, b):
      @pl.when(c < n_full)
      def _():
        pltpu.make_async_copy(
            tab_t.at[:, pl.ds(c * _CHUNK, _CHUNK)], bufs[b], sems[b]).wait()

    start(w, 0)
    start(w + _NW, 1)

    @pl.loop(0, (iters + 1) // 2)
    def _(i0):
      for b in range(2):
        i = i0 * 2 + b
        c = w + i * _NW
        finish(c, b)

        @pl.when(c < n_full)
        def _():
          _transpose_block(bufs[b], buft, _CHUNK)
          pltpu.sync_copy(buft, out_rm.at[pl.ds(c * _CHUNK, _CHUNK), :])

        start(c + 2 * _NW, b)

    @pl.when(w == _NW - 1)
    def _():
      pltpu.sync_copy(tab_t.at[:, pl.ds(n_full * _CHUNK, tail)], tbuf)
      _transpose_block(tbuf, tbuft, tail)
      pltpu.sync_copy(tbuft, out_rm.at[pl.ds(n_full * _CHUNK, tail), :])

  return pl.kernel(
      body,
      out_type=jax.ShapeDtypeStruct((vocab, _PITCH), jnp.float32),
      mesh=mesh,
      scratch_types=[
          pltpu.VMEM((emb_dim, _CHUNK), jnp.float32),
          pltpu.VMEM((emb_dim, _CHUNK), jnp.float32),
          pltpu.VMEM((_CHUNK, _PITCH), jnp.float32),
          pltpu.VMEM((emb_dim, tail), jnp.float32),
          pltpu.VMEM((tail, _PITCH), jnp.float32),
          pltpu.SemaphoreType.DMA,
          pltpu.SemaphoreType.DMA,
      ],
      compiler_params=pltpu.CompilerParams(
          use_tc_tiling_on_sc=True, needs_layout_passes=False),
  )


def _build_gather(seq, batch, emb_dim):
  n_chunks = seq
  mesh = plsc.VectorSubcoreMesh(core_axis_name="c", subcore_axis_name="s")

  def body(idx_hbm, table_hbm, out_hbm, idx_v, buf0, buf1, buft, sem0, sem1):
    w = _worker_id()
    bufs = (buf0, buf1)
    sems = (sem0, sem1)
    pltpu.sync_copy(idx_hbm.at[:, pl.ds(w * _CHUNK, _CHUNK)], idx_v)

    def start(s, b):
      pltpu.async_copy(table_hbm.at[idx_v.at[s]], bufs[b], sems[b])

    def finish(s, b):
      pltpu.make_async_copy(table_hbm.at[idx_v.at[s]], bufs[b], sems[b]).wait()

    jvs = [_iota16() + g * _LANES for g in range(8)]

    def emit(s, b):
      @plsc.parallel_loop(0, emb_dim, unroll=8)
      def _(d):
        dv = jnp.full((_LANES,), d, jnp.int32)
        for g in range(8):
          v = plsc.load_gather(bufs[b], [jvs[g], dv])
          buft[d >> 3, d & 7, pl.ds(g * _LANES, _LANES)] = v

      pltpu.sync_copy(buft, out_hbm.at[s, :, w])

    start(0, 0)
    start(1, 1)

    @pl.loop(0, n_chunks // 2 - 1)
    def _(s0):
      for b in range(2):
        s = s0 * 2 + b
        finish(s, b)
        emit(s, b)
        start(s + 2, b)

    for b in range(2):
      s = n_chunks - 2 + b
      finish(s, b)
      emit(s, b)

  return pl.kernel(
      body,
      out_type=jax.ShapeDtypeStruct(
          (seq, emb_dim // 8, batch // _CHUNK, 8, _CHUNK), jnp.float32),
      mesh=mesh,
      scratch_types=[
          pltpu.VMEM((seq, _CHUNK), jnp.int32),
          pltpu.VMEM((_CHUNK, _PITCH), jnp.float32),
          pltpu.VMEM((_CHUNK, _PITCH), jnp.float32),
          pltpu.VMEM((emb_dim // 8, 8, _CHUNK), jnp.float32),
          pltpu.SemaphoreType.DMA,
          pltpu.SemaphoreType.DMA,
      ],
      compiler_params=pltpu.CompilerParams(
          use_tc_tiling_on_sc=False, needs_layout_passes=False),
  )


def kernel(indices, table):
  batch, seq = indices.shape
  vocab, emb_dim = table.shape
  table_rm = _build_transpose(vocab, emb_dim)(table.T)
  idx_t = indices.T.astype(jnp.int32)
  out5 = _build_gather(seq, batch, emb_dim)(idx_t, table_rm)
  return out5.transpose(2, 4, 0, 1, 3).reshape(batch, seq, emb_dim)


# speedup vs baseline: 1.0227x; 1.0227x over previous
import jax
import jax.numpy as jnp
from jax import lax
from jax.experimental import pallas as pl
from jax.experimental.pallas import tpu as pltpu
from jax.experimental.pallas import tpu_sc as plsc

_NC = 2
_NS = 16
_NW = _NC * _NS
_LANES = 16

_CHUNK = 128
_NBUF = 4


def _worker_id():
  return lax.axis_index("s") * _NC + lax.axis_index("c")


def _iota16():
  return lax.iota(jnp.int32, _LANES)


def _build_transpose(vocab, emb_dim):
  n_full = vocab // _CHUNK
  tail = vocab - n_full * _CHUNK
  iters = n_full // _NW + 2
  iters += (-iters) % _NBUF
  mesh = plsc.VectorSubcoreMesh(core_axis_name="c", subcore_axis_name="s")

  def body(tab_t, out_rm, bufs, bufts, tbuf, tbuft, sems, wsems):
    w = _worker_id()
    iotas = [_iota16() + g * _LANES for g in range(4)]

    def transpose_block(src, dst, n_rows):
      @plsc.parallel_loop(0, n_rows, unroll=8)
      def _(j):
        jv = jnp.full((_LANES,), j, jnp.int32)
        for g in range(4):
          v = plsc.load_gather(src, [iotas[g], jv])
          dst[j, pl.ds(g * _LANES, _LANES)] = v

    def start_in(c, b):
      @pl.when(c < n_full)
      def _():
        pltpu.async_copy(
            tab_t.at[:, pl.ds(c * _CHUNK, _CHUNK)], bufs.at[b], sems.at[b])

    def wait_in(c, b):
      @pl.when(c < n_full)
      def _():
        pltpu.make_async_copy(
            tab_t.at[:, pl.ds(c * _CHUNK, _CHUNK)], bufs.at[b], sems.at[b]
        ).wait()

    def wait_out(c, wb):
      @pl.when(jnp.logical_and(c >= 0, c < n_full))
      def _():
        pltpu.make_async_copy(
            bufts.at[wb], out_rm.at[pl.ds(c * _CHUNK, _CHUNK), :], wsems.at[wb]
        ).wait()

    for b in range(_NBUF):
      start_in(w + b * _NW, b)

    @pl.loop(0, iters // _NBUF)
    def _(i0):
      for b in range(_NBUF):
        i = i0 * _NBUF + b
        c = w + i * _NW
        wb = i % 2
        wait_in(c, b)
        wait_out(c - _NW, (i - 1) % 2)

        @pl.when(c < n_full)
        def _():
          transpose_block(bufs.at[b], bufts.at[wb], _CHUNK)
          pltpu.async_copy(
              bufts.at[wb], out_rm.at[pl.ds(c * _CHUNK, _CHUNK), :],
              wsems.at[wb])

        start_in(c + _NBUF * _NW, b)

    wait_out(w + (iters - 1) * _NW - _NW, (iters - 1) % 2)

    @pl.when(w == _NW - 1)
    def _():
      pltpu.sync_copy(tab_t.at[:, pl.ds(n_full * _CHUNK, tail)], tbuf)
      transpose_block(tbuf, tbuft, tail)
      pltpu.sync_copy(tbuft, out_rm.at[pl.ds(n_full * _CHUNK, tail), :])

  return pl.kernel(
      body,
      out_type=jax.ShapeDtypeStruct((vocab, emb_dim), jnp.float32),
      mesh=mesh,
      scratch_types=[
          pltpu.VMEM((_NBUF, emb_dim, _CHUNK), jnp.float32),
          pltpu.VMEM((2, _CHUNK, emb_dim), jnp.float32),
          pltpu.VMEM((emb_dim, tail), jnp.float32),
          pltpu.VMEM((tail, emb_dim), jnp.float32),
          pltpu.SemaphoreType.DMA((_NBUF,)),
          pltpu.SemaphoreType.DMA((2,)),
      ],
      compiler_params=pltpu.CompilerParams(
          use_tc_tiling_on_sc=True, needs_layout_passes=False),
  )


def _build_gather(seq, batch, emb_dim):
  n_chunks = seq
  mesh = plsc.VectorSubcoreMesh(core_axis_name="c", subcore_axis_name="s")

  def body(idx_hbm, table_hbm, out_hbm, idx_v, bufs, bufts, sems, wsems):
    w = _worker_id()
    pltpu.sync_copy(idx_hbm.at[:, pl.ds(w * _CHUNK, _CHUNK)], idx_v)

    iotas = [_iota16() + g * _LANES for g in range(8)]

    def start(s, b):
      pltpu.async_copy(table_hbm.at[idx_v.at[s]], bufs.at[b], sems.at[b])

    def finish(s, b):
      pltpu.make_async_copy(
          table_hbm.at[idx_v.at[s]], bufs.at[b], sems.at[b]).wait()

    def wait_out(s, wb):
      pltpu.make_async_copy(
          bufts.at[wb], out_hbm.at[s, :, w], wsems.at[wb]).wait()

    def emit(s, b, wb):
      @plsc.parallel_loop(0, emb_dim, unroll=8)
      def _(d):
        dv = jnp.full((_LANES,), d, jnp.int32)
        for g in range(8):
          v = plsc.load_gather(bufs.at[b], [iotas[g], dv])
          bufts[wb, d >> 3, d & 7, pl.ds(g * _LANES, _LANES)] = v

      pltpu.async_copy(bufts.at[wb], out_hbm.at[s, :, w], wsems.at[wb])

    for b in range(_NBUF):
      start(b, b)

    @pl.loop(0, n_chunks // _NBUF - 1)
    def _(s0):
      for b in range(_NBUF):
        s = s0 * _NBUF + b
        finish(s, b)

        @pl.when(s >= 1)
        def _():
          wait_out(s - 1, (s - 1) % 2)

        emit(s, b, s % 2)
        start(s + _NBUF, b)

    for b in range(_NBUF):
      s = n_chunks - _NBUF + b
      finish(s, b)
      wait_out(s - 1, (s - 1) % 2)
      emit(s, b, s % 2)
    wait_out(n_chunks - 1, (n_chunks - 1) % 2)

  return pl.kernel(
      body,
      out_type=jax.ShapeDtypeStruct(
          (seq, emb_dim // 8, batch // _CHUNK, 8, _CHUNK), jnp.float32),
      mesh=mesh,
      scratch_types=[
          pltpu.VMEM((seq, _CHUNK), jnp.int32),
          pltpu.VMEM((_NBUF, _CHUNK, emb_dim), jnp.float32),
          pltpu.VMEM((2, emb_dim // 8, 8, _CHUNK), jnp.float32),
          pltpu.SemaphoreType.DMA((_NBUF,)),
          pltpu.SemaphoreType.DMA((2,)),
      ],
      compiler_params=pltpu.CompilerParams(
          use_tc_tiling_on_sc=False, needs_layout_passes=False),
  )


def kernel(indices, table):
  batch, seq = indices.shape
  vocab, emb_dim = table.shape
  table_rm = _build_transpose(vocab, emb_dim)(table.T)
  idx_t = indices.T.astype(jnp.int32)
  out5 = _build_gather(seq, batch, emb_dim)(idx_t, table_rm)
  return out5.transpose(2, 4, 0, 1, 3).reshape(batch, seq, emb_dim)


# speedup vs baseline: 1.1953x; 1.1688x over previous
import jax
import jax.numpy as jnp
from jax import lax
from jax.experimental import pallas as pl
from jax.experimental.pallas import tpu as pltpu
from jax.experimental.pallas import tpu_sc as plsc

_NC = 2
_NS = 16
_NW = _NC * _NS
_LANES = 16

_CHUNK = 128
_NBUF = 4
_PITCH = 72


def _worker_id():
  return lax.axis_index("s") * _NC + lax.axis_index("c")


def _iota16():
  return lax.iota(jnp.int32, _LANES)


def _build_transpose(vocab, emb_dim):
  n_full = vocab // _CHUNK
  tail = vocab - n_full * _CHUNK
  iters = n_full // _NW + 2
  iters += (-iters) % _NBUF
  mesh = plsc.VectorSubcoreMesh(core_axis_name="c", subcore_axis_name="s")

  def body(tab_t, out_rm, bufs, bufts, tbuf, tbuft, sems, wsems):
    w = _worker_id()
    def transpose_block(src, dst, n_rows):
      rows = [_iota16() + g * _LANES for g in range(n_rows // _LANES)]

      @plsc.parallel_loop(0, 64, unroll=8)
      def _(d):
        dv = jnp.full((_LANES,), d, jnp.int32)
        for g in range(n_rows // _LANES):
          v = src[d, pl.ds(g * _LANES, _LANES)]
          plsc.store_scatter(dst, [rows[g], dv], v)

    def start_in(c, b):
      @pl.when(c < n_full)
      def _():
        pltpu.async_copy(
            tab_t.at[:, pl.ds(c * _CHUNK, _CHUNK)], bufs.at[b], sems.at[b])

    def wait_in(c, b):
      @pl.when(c < n_full)
      def _():
        pltpu.make_async_copy(
            tab_t.at[:, pl.ds(c * _CHUNK, _CHUNK)], bufs.at[b], sems.at[b]
        ).wait()

    def wait_out(c, wb):
      @pl.when(jnp.logical_and(c >= 0, c < n_full))
      def _():
        pltpu.make_async_copy(
            bufts.at[wb], out_rm.at[pl.ds(c * _CHUNK, _CHUNK), :], wsems.at[wb]
        ).wait()

    for b in range(_NBUF):
      start_in(w + b * _NW, b)

    @pl.loop(0, iters // _NBUF)
    def _(i0):
      for b in range(_NBUF):
        i = i0 * _NBUF + b
        c = w + i * _NW
        wb = i % 2
        wait_in(c, b)
        wait_out(c - _NW, (i - 1) % 2)

        @pl.when(c < n_full)
        def _():
          transpose_block(bufs.at[b], bufts.at[wb], _CHUNK)
          pltpu.async_copy(
              bufts.at[wb], out_rm.at[pl.ds(c * _CHUNK, _CHUNK), :],
              wsems.at[wb])

        start_in(c + _NBUF * _NW, b)

    wait_out(w + (iters - 1) * _NW - _NW, (iters - 1) % 2)

    @pl.when(w == _NW - 1)
    def _():
      pltpu.sync_copy(tab_t.at[:, pl.ds(n_full * _CHUNK, tail)], tbuf)
      transpose_block(tbuf, tbuft, tail)
      pltpu.sync_copy(tbuft, out_rm.at[pl.ds(n_full * _CHUNK, tail), :])

  return pl.kernel(
      body,
      out_type=jax.ShapeDtypeStruct((vocab, _PITCH), jnp.float32),
      mesh=mesh,
      scratch_types=[
          pltpu.VMEM((_NBUF, emb_dim, _CHUNK), jnp.float32),
          pltpu.VMEM((2, _CHUNK, _PITCH), jnp.float32),
          pltpu.VMEM((emb_dim, tail), jnp.float32),
          pltpu.VMEM((tail, _PITCH), jnp.float32),
          pltpu.SemaphoreType.DMA((_NBUF,)),
          pltpu.SemaphoreType.DMA((2,)),
      ],
      compiler_params=pltpu.CompilerParams(
          use_tc_tiling_on_sc=True, needs_layout_passes=False),
  )


def _build_gather(seq, batch, emb_dim):
  n_chunks = seq
  mesh = plsc.VectorSubcoreMesh(core_axis_name="c", subcore_axis_name="s")

  def body(idx_hbm, table_hbm, out_hbm, idx_v, bufs, bufts, sems, wsems):
    w = _worker_id()
    pltpu.sync_copy(idx_hbm.at[:, pl.ds(w * _CHUNK, _CHUNK)], idx_v)

    iotas = [_iota16() + g * _LANES for g in range(8)]

    def start(s, b):
      pltpu.async_copy(table_hbm.at[idx_v.at[s]], bufs.at[b], sems.at[b])

    def finish(s, b):
      pltpu.make_async_copy(
          table_hbm.at[idx_v.at[s]], bufs.at[b], sems.at[b]).wait()

    def wait_out(s, wb):
      pltpu.make_async_copy(
          bufts.at[wb], out_hbm.at[s, :, w], wsems.at[wb]).wait()

    def emit(s, b, wb):
      @plsc.parallel_loop(0, emb_dim, unroll=8)
      def _(d):
        dv = jnp.full((_LANES,), d, jnp.int32)
        for g in range(8):
          v = plsc.load_gather(bufs.at[b], [iotas[g], dv])
          bufts[wb, d >> 3, d & 7, pl.ds(g * _LANES, _LANES)] = v

      pltpu.async_copy(bufts.at[wb], out_hbm.at[s, :, w], wsems.at[wb])

    for b in range(_NBUF):
      start(b, b)

    @pl.loop(0, n_chunks // _NBUF - 1)
    def _(s0):
      for b in range(_NBUF):
        s = s0 * _NBUF + b
        finish(s, b)

        @pl.when(s >= 1)
        def _():
          wait_out(s - 1, (s - 1) % 2)

        emit(s, b, s % 2)
        start(s + _NBUF, b)

    for b in range(_NBUF):
      s = n_chunks - _NBUF + b
      finish(s, b)
      wait_out(s - 1, (s - 1) % 2)
      emit(s, b, s % 2)
    wait_out(n_chunks - 1, (n_chunks - 1) % 2)

  return pl.kernel(
      body,
      out_type=jax.ShapeDtypeStruct(
          (seq, emb_dim // 8, batch // _CHUNK, 8, _CHUNK), jnp.float32),
      mesh=mesh,
      scratch_types=[
          pltpu.VMEM((seq, _CHUNK), jnp.int32),
          pltpu.VMEM((_NBUF, _CHUNK, _PITCH), jnp.float32),
          pltpu.VMEM((2, emb_dim // 8, 8, _CHUNK), jnp.float32),
          pltpu.SemaphoreType.DMA((_NBUF,)),
          pltpu.SemaphoreType.DMA((2,)),
      ],
      compiler_params=pltpu.CompilerParams(
          use_tc_tiling_on_sc=False, needs_layout_passes=False),
  )


def kernel(indices, table):
  batch, seq = indices.shape
  vocab, emb_dim = table.shape
  table_rm = _build_transpose(vocab, emb_dim)(table.T)
  idx_t = indices.T.astype(jnp.int32)
  out5 = _build_gather(seq, batch, emb_dim)(idx_t, table_rm)
  return out5.transpose(2, 4, 0, 1, 3).reshape(batch, seq, emb_dim)


# speedup vs baseline: 1.6862x; 1.4107x over previous
import jax
import jax.numpy as jnp
from jax import lax
from jax.experimental import pallas as pl
from jax.experimental.pallas import tpu as pltpu
from jax.experimental.pallas import tpu_sc as plsc

_NC = 2
_NS = 16
_NW = _NC * _NS
_CHUNK = 128


def _build(num_rows, emb_dim):
  rows_per_w = num_rows // _NW
  n_chunks = rows_per_w // _CHUNK
  mesh = plsc.VectorSubcoreMesh(core_axis_name="c", subcore_axis_name="s")

  def body(idx_hbm, table_hbm, out_hbm, idx_v, buf0, buf1, sem0, sem1):
    wid = lax.axis_index("s") * _NC + lax.axis_index("c")
    base = wid * rows_per_w
    pltpu.sync_copy(idx_hbm.at[wid], idx_v)
    bufs = (buf0, buf1)
    sems = (sem0, sem1)

    def start(chunk, b):
      pltpu.async_copy(table_hbm.at[idx_v.at[chunk]], bufs[b], sems[b])

    def finish(chunk, b):
      pltpu.make_async_copy(table_hbm.at[idx_v.at[chunk]], bufs[b], sems[b]).wait()

    start(0, 0)
    start(1, 1)

    @pl.loop(0, n_chunks // 2 - 1)
    def _(g0):
      for b in range(2):
        g = g0 * 2 + b
        finish(g, b)
        pltpu.sync_copy(bufs[b], out_hbm.at[pl.ds(base + g * _CHUNK, _CHUNK)])
        start(g + 2, b)

    for b in range(2):
      g = n_chunks - 2 + b
      finish(g, b)
      pltpu.sync_copy(bufs[b], out_hbm.at[pl.ds(base + g * _CHUNK, _CHUNK)])

  return pl.kernel(
      body,
      out_type=jax.ShapeDtypeStruct((num_rows, emb_dim), jnp.float32),
      mesh=mesh,
      scratch_types=[
          pltpu.VMEM((n_chunks, _CHUNK), jnp.int32),
          pltpu.VMEM((_CHUNK, emb_dim), jnp.float32),
          pltpu.VMEM((_CHUNK, emb_dim), jnp.float32),
          pltpu.SemaphoreType.DMA,
          pltpu.SemaphoreType.DMA,
      ],
      compiler_params=pltpu.CompilerParams(use_tc_tiling_on_sc=False),
  )


def kernel(indices, table):
  batch, seq = indices.shape
  vocab, emb_dim = table.shape
  num_rows = batch * seq
  idx3 = indices.astype(jnp.int32).reshape(_NW, num_rows // (_NW * _CHUNK), _CHUNK)
  out = _build(num_rows, emb_dim)(idx3, table)
  return out.reshape(batch, seq, emb_dim)
